# baseline (speedup 1.0000x reference)
import functools

import jax
import jax.numpy as jnp
from jax.experimental import pallas as pl
from jax.experimental.pallas import tpu as pltpu

PB = 1024
TB = 128
RU = 64


def _body(ptsT_ref, gb_ref, gbT_ref,
          cls_ref, plx_ref, ply_ref, plz_ref, bidx_ref,
          cxw, cyw, czw, ccw, ssw, hxw, hyw, hzw, iacw, wg_ref):
    R = gb_ref.shape[0]
    M = 128
    nk = R // M

    @pl.when(pl.program_id(0) == 0)
    def _prep():
        gb = gb_ref[...]
        ang = -gb[:, 6:7]
        c = jnp.cos(ang)
        s = jnp.sin(ang)
        valid = (gb[:, 3:4] + gb[:, 4:5] + gb[:, 5:6]) > 0.0
        hx = jnp.where(valid, gb[:, 3:4] * 0.5, -1.0)
        hy = gb[:, 4:5] * 0.5
        hz = gb[:, 5:6] * 0.5
        loc = jax.lax.broadcasted_iota(jnp.int32, (R, 1), 0) % M
        iac = (loc * 4).astype(jnp.float32) + gb[:, 7:8]
        for ref, col in ((cxw, gb[:, 0:1]), (cyw, gb[:, 1:2]),
                         (czw, gb[:, 2:3]), (ccw, c), (ssw, s),
                         (hxw, hx), (hyw, hy), (hzw, hz), (iacw, iac)):
            ref[...] = jnp.broadcast_to(col, (R, TB))
        gbT = gbT_ref[...]
        angT = -gbT[6:7, :]
        wg_ref[0:3, :] = gbT[0:3, :]
        wg_ref[3:4, :] = jnp.cos(angT)
        wg_ref[4:5, :] = jnp.sin(angT)
        wg_ref[5:8, :] = gbT[3:6, :]

    blk = ptsT_ref[...]
    pb = blk.shape[1]
    nt = pb // TB
    nu = R // RU
    upc = M // RU
    sent = jnp.float32(4 * M * nk)

    cls_rows, plx_rows, ply_rows, plz_rows, bidx_rows = [], [], [], [], []
    for t in range(nt):
        tsl = slice(t * TB, (t + 1) * TB)
        bs = blk[0:1, tsl]
        xr = blk[1:2, tsl]
        yr = blk[2:3, tsl]
        zr = blk[3:4, tsl]

        def unit(u, fif):
            rb = u * RU
            rs = pl.ds(rb, RU)
            kf = (u // upc).astype(jnp.float32)
            dx = xr - cxw[rs, :]
            dy = yr - cyw[rs, :]
            cc = ccw[rs, :]
            ss = ssw[rs, :]
            lx = dx * cc - dy * ss
            ly = dx * ss + dy * cc
            dz = zr - czw[rs, :]
            inb = ((jnp.abs(lx) <= hxw[rs, :])
                   & (jnp.abs(ly) <= hyw[rs, :])
                   & (jnp.abs(dz) <= hzw[rs, :]))
            cand = jnp.where(inb, iacw[rs, :], sent)
            mn = jnp.min(cand, axis=0, keepdims=True)
            mng = mn + kf * jnp.float32(4 * M)
            return jnp.where(bs == kf, jnp.minimum(fif, mng), fif)

        fif = jax.lax.fori_loop(0, nu, unit, jnp.full((1, TB), sent),
                                unroll=1)
        fg = fif < sent
        fii = fif.astype(jnp.int32)
        fi = jax.lax.shift_right_logical(fii, 2)

        G = jnp.zeros((8, TB), jnp.float32)
        for k in range(nk):
            ohf = (iacw[k * M:(k + 1) * M, :]
                   == (fif - jnp.float32(k * 4 * M))).astype(jnp.float32)
            G = G + jax.lax.dot_general(
                wg_ref[:, k * M:(k + 1) * M], ohf, (((1,), (0,)), ((), ())),
                precision=jax.lax.Precision.HIGHEST,
                preferred_element_type=jnp.float32)

        px = xr - G[0:1, :]
        py = yr - G[1:2, :]
        pz = zr - G[2:3, :]
        gc = G[3:4, :]
        gs = G[4:5, :]
        rx = px * gc - py * gs
        ry = px * gs + py * gc
        cls_rows.append(jnp.where(fg, (fii & 3) + 1, 0))
        plx_rows.append(jnp.where(fg, rx / G[5:6, :] + 0.5, 0.0))
        ply_rows.append(jnp.where(fg, ry / G[6:7, :] + 0.5, 0.0))
        plz_rows.append(jnp.where(fg, pz / G[7:8, :] + 0.5, 0.0))
        bidx_rows.append(jnp.where(fg, fi, -1))

    cls = jnp.concatenate(cls_rows, axis=1)
    plx = jnp.concatenate(plx_rows, axis=1)
    ply = jnp.concatenate(ply_rows, axis=1)
    plz = jnp.concatenate(plz_rows, axis=1)
    bidx = jnp.concatenate(bidx_rows, axis=1)

    cls_ref[...] = cls.reshape(1, 1, cls.shape[-1])
    plx_ref[...] = plx.reshape(1, 1, plx.shape[-1])
    ply_ref[...] = ply.reshape(1, 1, ply.shape[-1])
    plz_ref[...] = plz.reshape(1, 1, plz.shape[-1])
    bidx_ref[...] = bidx.reshape(1, 1, bidx.shape[-1])


@functools.partial(jax.jit, static_argnames=())
def kernel(points, gt_boxes):
    n = points.shape[0]
    b, m, _ = gt_boxes.shape
    r = b * m
    pb = PB if n % PB == 0 else n
    g = n // pb

    ptsT = jnp.transpose(points)
    gb = gt_boxes.reshape(r, 8)
    gbT = jnp.transpose(gb)

    grid = (g,)
    out_shapes = [
        jax.ShapeDtypeStruct((g, 1, pb), jnp.int32),
        jax.ShapeDtypeStruct((g, 1, pb), jnp.float32),
        jax.ShapeDtypeStruct((g, 1, pb), jnp.float32),
        jax.ShapeDtypeStruct((g, 1, pb), jnp.float32),
        jax.ShapeDtypeStruct((g, 1, pb), jnp.int32),
    ]
    out_specs = [pl.BlockSpec((1, 1, pb), lambda i: (i, 0, 0))
                 for _ in range(5)]
    in_specs = [
        pl.BlockSpec((4, pb), lambda i: (0, i)),
        pl.BlockSpec((r, 8), lambda i: (0, 0)),
        pl.BlockSpec((8, r), lambda i: (0, 0)),
    ]
    scratch = [pltpu.VMEM((r, TB), jnp.float32) for _ in range(9)]
    scratch.append(pltpu.VMEM((8, r), jnp.float32))
    cls, plx, ply, plz, bidx = pl.pallas_call(
        _body,
        grid=grid,
        in_specs=in_specs,
        out_specs=out_specs,
        out_shape=out_shapes,
        scratch_shapes=scratch,
    )(ptsT, gb, gbT)

    part = jnp.concatenate(
        [plx.reshape(n, 1), ply.reshape(n, 1), plz.reshape(n, 1)], axis=1)
    return cls.reshape(n), part, bidx.reshape(n)


# speedup vs baseline: 2.0448x; 2.0448x over previous
import functools

import jax
import jax.numpy as jnp
from jax.experimental import pallas as pl
from jax.experimental.pallas import tpu as pltpu

PB = 1024
TB = 256


def _body(ptsT_ref, gb_ref, gbT_ref,
          cls_ref, plx_ref, ply_ref, plz_ref, bidx_ref,
          prep_ref, wg_ref):
    R = gb_ref.shape[0]
    M = 128
    nk = R // M

    @pl.when(pl.program_id(0) == 0)
    def _prep():
        gb = gb_ref[...]
        ang = -gb[:, 6:7]
        c = jnp.cos(ang)
        s = jnp.sin(ang)
        valid = (gb[:, 3:4] + gb[:, 4:5] + gb[:, 5:6]) > 0.0
        hx = jnp.where(valid, gb[:, 3:4] * 0.5, -1.0)
        hy = gb[:, 4:5] * 0.5
        hz = gb[:, 5:6] * 0.5
        loc = jax.lax.broadcasted_iota(jnp.int32, (R, 1), 0) % M
        iac = (loc * 4).astype(jnp.float32) + gb[:, 7:8]
        prep_ref[...] = jnp.concatenate(
            [gb[:, 0:1], gb[:, 1:2], gb[:, 2:3], c, s, hx, hy, hz,
             iac, jnp.zeros((R, 7), jnp.float32)], axis=1)
        gbT = gbT_ref[...]
        angT = -gbT[6:7, :]
        wg_ref[0:3, :] = gbT[0:3, :]
        wg_ref[3:4, :] = jnp.cos(angT)
        wg_ref[4:5, :] = jnp.sin(angT)
        wg_ref[5:8, :] = gbT[3:6, :]

    pc = prep_ref[...]
    cx = pc[:, 0:1]
    cy = pc[:, 1:2]
    cz = pc[:, 2:3]
    cc = pc[:, 3:4]
    ss = pc[:, 4:5]
    hx = pc[:, 5:6]
    hy = pc[:, 6:7]
    hz = pc[:, 7:8]
    iac = pc[:, 8:9]

    blk = ptsT_ref[...]
    pb = blk.shape[1]
    nt = pb // TB
    ml4 = jnp.float32(4 * M)
    sent = jnp.float32(4 * M * nk)

    cls_rows, plx_rows, ply_rows, plz_rows, bidx_rows = [], [], [], [], []
    for t in range(nt):
        tsl = slice(t * TB, (t + 1) * TB)
        bs = blk[0:1, tsl]
        xr = blk[1:2, tsl]
        yr = blk[2:3, tsl]
        zr = blk[3:4, tsl]
        fis = []
        for k in range(nk):
            sl = slice(k * M, (k + 1) * M)
            dx = xr - cx[sl]
            dy = yr - cy[sl]
            lx = dx * cc[sl] - dy * ss[sl]
            ly = dx * ss[sl] + dy * cc[sl]
            dz = zr - cz[sl]
            inb = ((jnp.abs(lx) <= hx[sl]) & (jnp.abs(ly) <= hy[sl])
                   & (jnp.abs(dz) <= hz[sl]))
            cand = jnp.where(inb, iac[sl], sent)
            mn = jnp.min(cand, axis=0, keepdims=True)
            fis.append(jnp.where(mn < ml4, mn + (k * 4 * M), sent))
        fif = fis[nk - 1]
        for k in range(nk - 2, -1, -1):
            fif = jnp.where(bs == jnp.float32(k), fis[k], fif)
        fg = fif < sent
        fii = fif.astype(jnp.int32)
        fi = jax.lax.shift_right_logical(fii, 2)

        G = jnp.zeros((8, TB), jnp.float32)
        for k in range(nk):
            ohf = (iac[k * M:(k + 1) * M]
                   == (fif - jnp.float32(k * 4 * M))).astype(jnp.float32)
            G = G + jax.lax.dot_general(
                wg_ref[:, k * M:(k + 1) * M], ohf, (((1,), (0,)), ((), ())),
                precision=jax.lax.Precision.HIGHEST,
                preferred_element_type=jnp.float32)

        px = xr - G[0:1, :]
        py = yr - G[1:2, :]
        pz = zr - G[2:3, :]
        gc = G[3:4, :]
        gs = G[4:5, :]
        rx = px * gc - py * gs
        ry = px * gs + py * gc
        cls_rows.append(jnp.where(fg, (fii & 3) + 1, 0))
        plx_rows.append(jnp.where(fg, rx / G[5:6, :] + 0.5, 0.0))
        ply_rows.append(jnp.where(fg, ry / G[6:7, :] + 0.5, 0.0))
        plz_rows.append(jnp.where(fg, pz / G[7:8, :] + 0.5, 0.0))
        bidx_rows.append(jnp.where(fg, fi, -1))

    cls = jnp.concatenate(cls_rows, axis=1)
    plx = jnp.concatenate(plx_rows, axis=1)
    ply = jnp.concatenate(ply_rows, axis=1)
    plz = jnp.concatenate(plz_rows, axis=1)
    bidx = jnp.concatenate(bidx_rows, axis=1)

    cls_ref[...] = cls.reshape(1, 1, cls.shape[-1])
    plx_ref[...] = plx.reshape(1, 1, plx.shape[-1])
    ply_ref[...] = ply.reshape(1, 1, ply.shape[-1])
    plz_ref[...] = plz.reshape(1, 1, plz.shape[-1])
    bidx_ref[...] = bidx.reshape(1, 1, bidx.shape[-1])


@functools.partial(jax.jit, static_argnames=())
def kernel(points, gt_boxes):
    n = points.shape[0]
    b, m, _ = gt_boxes.shape
    r = b * m
    pb = PB if n % PB == 0 else n
    g = n // pb

    ptsT = jnp.transpose(points)
    gb = gt_boxes.reshape(r, 8)
    gbT = jnp.transpose(gb)

    grid = (g,)
    out_shapes = [
        jax.ShapeDtypeStruct((g, 1, pb), jnp.int32),
        jax.ShapeDtypeStruct((g, 1, pb), jnp.float32),
        jax.ShapeDtypeStruct((g, 1, pb), jnp.float32),
        jax.ShapeDtypeStruct((g, 1, pb), jnp.float32),
        jax.ShapeDtypeStruct((g, 1, pb), jnp.int32),
    ]
    out_specs = [pl.BlockSpec((1, 1, pb), lambda i: (i, 0, 0))
                 for _ in range(5)]
    in_specs = [
        pl.BlockSpec((4, pb), lambda i: (0, i)),
        pl.BlockSpec((r, 8), lambda i: (0, 0)),
        pl.BlockSpec((8, r), lambda i: (0, 0)),
    ]
    scratch = [
        pltpu.VMEM((r, 16), jnp.float32),
        pltpu.VMEM((8, r), jnp.float32),
    ]
    cls, plx, ply, plz, bidx = pl.pallas_call(
        _body,
        grid=grid,
        in_specs=in_specs,
        out_specs=out_specs,
        out_shape=out_shapes,
        scratch_shapes=scratch,
    )(ptsT, gb, gbT)

    part = jnp.concatenate(
        [plx.reshape(n, 1), ply.reshape(n, 1), plz.reshape(n, 1)], axis=1)
    return cls.reshape(n), part, bidx.reshape(n)


# speedup vs baseline: 2.2344x; 1.0927x over previous
import functools

import jax
import jax.numpy as jnp
from jax.experimental import pallas as pl
from jax.experimental.pallas import tpu as pltpu

PB = 1024
TB = 256


def _body(ptsT_ref, gb_ref, gbT_ref,
          cls_ref, plx_ref, ply_ref, plz_ref, bidx_ref,
          prep_ref, wg_ref):
    R = gb_ref.shape[0]
    M = 128
    nk = R // M

    @pl.when(pl.program_id(0) == 0)
    def _prep():
        gb = gb_ref[...]
        ang = -gb[:, 6:7]
        c = jnp.cos(ang)
        s = jnp.sin(ang)
        valid = (gb[:, 3:4] + gb[:, 4:5] + gb[:, 5:6]) > 0.0
        hx = jnp.where(valid, gb[:, 3:4] * 0.5, -1.0)
        hy = gb[:, 4:5] * 0.5
        hz = gb[:, 5:6] * 0.5
        prep_ref[...] = jnp.concatenate(
            [gb[:, 0:1], gb[:, 1:2], gb[:, 2:3], c, s, hx, hy, hz,
             jnp.zeros((R, 8), jnp.float32)], axis=1)
        gbT = gbT_ref[...]
        angT = -gbT[6:7, :]
        wg_ref[0:3, :] = gbT[0:3, :]
        wg_ref[3:4, :] = jnp.cos(angT)
        wg_ref[4:5, :] = jnp.sin(angT)
        wg_ref[5:8, :] = gbT[3:6, :]
        wg_ref[8:9, :] = gbT[7:8, :]
        wg_ref[9:16, :] = jnp.zeros((7, R), jnp.float32)

    pc = prep_ref[...]
    cx = pc[:, 0:1]
    cy = pc[:, 1:2]
    cz = pc[:, 2:3]
    cc = pc[:, 3:4]
    ss = pc[:, 4:5]
    hx = pc[:, 5:6]
    hy = pc[:, 6:7]
    hz = pc[:, 7:8]

    blk = ptsT_ref[...]
    pb = blk.shape[1]
    nt = pb // TB
    iota = jax.lax.broadcasted_iota(jnp.int32, (M, TB), 0).astype(jnp.float32)
    mlf = jnp.float32(M)
    sent = jnp.float32(M * nk)

    cls_rows, plx_rows, ply_rows, plz_rows, bidx_rows = [], [], [], [], []
    for t in range(nt):
        tsl = slice(t * TB, (t + 1) * TB)
        bs = blk[0:1, tsl]
        xr = blk[1:2, tsl]
        yr = blk[2:3, tsl]
        zr = blk[3:4, tsl]
        fis = []
        for k in range(nk):
            sl = slice(k * M, (k + 1) * M)
            dx = xr - cx[sl]
            dy = yr - cy[sl]
            lx = dx * cc[sl] - dy * ss[sl]
            ly = dx * ss[sl] + dy * cc[sl]
            dz = zr - cz[sl]
            inb = ((jnp.abs(lx) <= hx[sl]) & (jnp.abs(ly) <= hy[sl])
                   & (jnp.abs(dz) <= hz[sl]))
            cand = jnp.where(inb, iota, sent)
            mn = jnp.min(cand, axis=0, keepdims=True)
            fis.append(jnp.where(mn < mlf, mn + (k * M), sent))
        fif = fis[nk - 1]
        for k in range(nk - 2, -1, -1):
            fif = jnp.where(bs == jnp.float32(k), fis[k], fif)
        fg = fif < sent
        fi = fif.astype(jnp.int32)

        G = jnp.zeros((16, TB), jnp.float32)
        for k in range(nk):
            ohf = (iota == (fif - jnp.float32(k * M))).astype(jnp.float32)
            G = G + jax.lax.dot_general(
                wg_ref[:, k * M:(k + 1) * M], ohf, (((1,), (0,)), ((), ())),
                precision=jax.lax.Precision.HIGHEST,
                preferred_element_type=jnp.float32)

        px = xr - G[0:1, :]
        py = yr - G[1:2, :]
        pz = zr - G[2:3, :]
        gc = G[3:4, :]
        gs = G[4:5, :]
        rx = px * gc - py * gs
        ry = px * gs + py * gc
        cls_rows.append(jnp.where(fg, G[8:9, :].astype(jnp.int32) + 1, 0))
        plx_rows.append(jnp.where(fg, rx / G[5:6, :] + 0.5, 0.0))
        ply_rows.append(jnp.where(fg, ry / G[6:7, :] + 0.5, 0.0))
        plz_rows.append(jnp.where(fg, pz / G[7:8, :] + 0.5, 0.0))
        bidx_rows.append(jnp.where(fg, fi, -1))

    cls = jnp.concatenate(cls_rows, axis=1)
    plx = jnp.concatenate(plx_rows, axis=1)
    ply = jnp.concatenate(ply_rows, axis=1)
    plz = jnp.concatenate(plz_rows, axis=1)
    bidx = jnp.concatenate(bidx_rows, axis=1)

    cls_ref[...] = cls.reshape(1, 1, cls.shape[-1])
    plx_ref[...] = plx.reshape(1, 1, plx.shape[-1])
    ply_ref[...] = ply.reshape(1, 1, ply.shape[-1])
    plz_ref[...] = plz.reshape(1, 1, plz.shape[-1])
    bidx_ref[...] = bidx.reshape(1, 1, bidx.shape[-1])


@functools.partial(jax.jit, static_argnames=())
def kernel(points, gt_boxes):
    n = points.shape[0]
    b, m, _ = gt_boxes.shape
    r = b * m
    pb = PB if n % PB == 0 else n
    g = n // pb

    ptsT = jnp.transpose(points)
    gb = gt_boxes.reshape(r, 8)
    gbT = jnp.transpose(gb)

    grid = (g,)
    out_shapes = [
        jax.ShapeDtypeStruct((g, 1, pb), jnp.int32),
        jax.ShapeDtypeStruct((g, 1, pb), jnp.float32),
        jax.ShapeDtypeStruct((g, 1, pb), jnp.float32),
        jax.ShapeDtypeStruct((g, 1, pb), jnp.float32),
        jax.ShapeDtypeStruct((g, 1, pb), jnp.int32),
    ]
    out_specs = [pl.BlockSpec((1, 1, pb), lambda i: (i, 0, 0))
                 for _ in range(5)]
    in_specs = [
        pl.BlockSpec((4, pb), lambda i: (0, i)),
        pl.BlockSpec((r, 8), lambda i: (0, 0)),
        pl.BlockSpec((8, r), lambda i: (0, 0)),
    ]
    scratch = [
        pltpu.VMEM((r, 16), jnp.float32),
        pltpu.VMEM((16, r), jnp.float32),
    ]
    cls, plx, ply, plz, bidx = pl.pallas_call(
        _body,
        grid=grid,
        in_specs=in_specs,
        out_specs=out_specs,
        out_shape=out_shapes,
        scratch_shapes=scratch,
    )(ptsT, gb, gbT)

    part = jnp.concatenate(
        [plx.reshape(n, 1), ply.reshape(n, 1), plz.reshape(n, 1)], axis=1)
    return cls.reshape(n), part, bidx.reshape(n)
